# Initial kernel scaffold; baseline (speedup 1.0000x reference)
import jax
import jax.numpy as jnp
from jax.experimental import pallas as pl


def kernel(x):
    raise NotImplementedError("write your pallas kernel here")


# speedup vs baseline: 12.0582x; 12.0582x over previous
import jax
import jax.numpy as jnp
from jax.experimental import pallas as pl
from jax.experimental.pallas import tpu as pltpu

_C1 = 1.5
_C2 = 3.0
_W = 1.5


def _tri(v, center):
    return jnp.maximum(1.0 - jnp.abs(v - center) / _W, 0.0)


def _fuzzy_kernel(a_ref, b_ref, o_ref):
    a = a_ref[...]
    b = b_ref[...]
    blk, ho, w = a.shape
    wo = w // 2

    m1a, m1b = _tri(a, _C1), _tri(b, _C1)
    m2a, m2b = _tri(a, _C2), _tri(b, _C2)

    t_s1 = m1a + m1b
    t_s2 = m2a + m2b
    t_d1 = m1a * a + m1b * b
    t_d2 = m2a * a + m2b * b
    t_n1 = m1a * a * a + m1b * b * b
    t_n2 = m2a * a * a + m2b * b * b

    ki = jax.lax.broadcasted_iota(jnp.int32, (w, wo), 0)
    ji = jax.lax.broadcasted_iota(jnp.int32, (w, wo), 1)
    pair = (ki // 2 == ji).astype(jnp.float32)

    def pool(t):
        t2 = t.reshape(blk * ho, w)
        r = jax.lax.dot(t2, pair, precision=jax.lax.Precision.HIGHEST)
        return r.reshape(blk, ho, wo)

    s1, s2 = pool(t_s1), pool(t_s2)
    d1, d2 = pool(t_d1), pool(t_d2)
    n1, n2 = pool(t_n1), pool(t_n2)

    use1 = s1 >= s2
    num = jnp.where(use1, n1, n2)
    den = jnp.where(use1, d1, d2)
    o_ref[...] = jnp.where(den == 0.0, 0.0, num / jnp.where(den == 0.0, 1.0, den))


def kernel(x):
    B, C, H, W = x.shape
    Ho, Wo = H // 2, W // 2
    BLK = 8
    n = B * C
    xv = x.reshape(n, Ho, 2 * W)
    out = pl.pallas_call(
        _fuzzy_kernel,
        grid=(n // BLK,),
        in_specs=[
            pl.BlockSpec((BLK, Ho, W), lambda i: (i, 0, 0)),
            pl.BlockSpec((BLK, Ho, W), lambda i: (i, 0, 1)),
        ],
        out_specs=pl.BlockSpec((BLK, Ho, Wo), lambda i: (i, 0, 0)),
        out_shape=jax.ShapeDtypeStruct((n, Ho, Wo), x.dtype),
        compiler_params=pltpu.CompilerParams(dimension_semantics=("parallel",)),
    )(xv, xv)
    return out.reshape(B, C, Ho, Wo)


# speedup vs baseline: 15.9852x; 1.3257x over previous
import jax
import jax.numpy as jnp
from jax.experimental import pallas as pl
from jax.experimental.pallas import tpu as pltpu

_C1 = 1.5
_C2 = 3.0
_W = 1.5


def _tri(v, center):
    return jnp.maximum(1.0 - jnp.abs(v - center) / _W, 0.0)


def _fuzzy_kernel(x_ref, o_ref):
    x = x_ref[...]
    blk, ho, w2 = x.shape
    w = w2 // 2
    wo = w // 2
    a = x[:, :, :w]
    b = x[:, :, w:]

    m1a, m1b = _tri(a, _C1), _tri(b, _C1)
    m2a, m2b = _tri(a, _C2), _tri(b, _C2)

    t_s1 = m1a + m1b
    t_s2 = m2a + m2b
    t_d1 = m1a * a + m1b * b
    t_d2 = m2a * a + m2b * b
    t_n1 = m1a * a * a + m1b * b * b
    t_n2 = m2a * a * a + m2b * b * b

    def pairsum(t):
        return t + jnp.roll(t, -1, axis=-1)

    s1 = pairsum(t_s1)
    s2 = pairsum(t_s2)
    u = jnp.where(s1 >= s2, 1.0, 0.0)
    parity = jax.lax.broadcasted_iota(jnp.int32, u.shape, 2) % 2
    u = jnp.where(parity == 0, u, jnp.roll(u, 1, axis=-1))
    v = 1.0 - u
    num_f = pairsum(u * t_n1 + v * t_n2)
    den_f = pairsum(u * t_d1 + v * t_d2)

    ki = jax.lax.broadcasted_iota(jnp.int32, (w, wo), 0)
    ji = jax.lax.broadcasted_iota(jnp.int32, (w, wo), 1)
    keep = (ki == 2 * ji).astype(jnp.float32)
    stacked = jnp.concatenate(
        [num_f.reshape(blk * ho, w), den_f.reshape(blk * ho, w)], axis=0)
    r = jax.lax.dot(stacked, keep, precision=jax.lax.Precision.HIGHEST)
    num = r[: blk * ho].reshape(blk, ho, wo)
    den = r[blk * ho:].reshape(blk, ho, wo)

    o_ref[...] = jnp.where(den == 0.0, 0.0, num / jnp.where(den == 0.0, 1.0, den))


def kernel(x):
    B, C, H, W = x.shape
    Ho, Wo = H // 2, W // 2
    BLK = 8
    n = B * C
    xv = x.reshape(n, Ho, 2 * W)
    out = pl.pallas_call(
        _fuzzy_kernel,
        grid=(n // BLK,),
        in_specs=[pl.BlockSpec((BLK, Ho, 2 * W), lambda i: (i, 0, 0))],
        out_specs=pl.BlockSpec((BLK, Ho, Wo), lambda i: (i, 0, 0)),
        out_shape=jax.ShapeDtypeStruct((n, Ho, Wo), x.dtype),
        compiler_params=pltpu.CompilerParams(dimension_semantics=("parallel",)),
    )(xv)
    return out.reshape(B, C, Ho, Wo)
